# baseline (speedup 1.0000x reference)
import functools

import jax
import jax.numpy as jnp
from jax import lax
from jax.experimental import pallas as pl
from jax.experimental.pallas import tpu as pltpu
from jax.experimental.pallas import tpu_sc as plsc

B = 16384
D = 64
NC = 2
NS = 16
NW = NC * NS
BPW = B // NW
CH = 128
NCH = BPW // CH
L = 16
PAD = L + 1

_mesh = plsc.VectorSubcoreMesh(core_axis_name="c", subcore_axis_name="s")


@functools.partial(
    pl.kernel,
    out_type=jax.ShapeDtypeStruct((B,), jnp.float32),
    mesh=_mesh,
    compiler_params=pltpu.CompilerParams(
        needs_layout_passes=False, use_tc_tiling_on_sc=False
    ),
    scratch_types=[
        pltpu.VMEM((NCH, CH), jnp.int32),
        pltpu.VMEM((NCH, CH), jnp.int32),
        pltpu.VMEM((BPW, D), jnp.float32),
        pltpu.VMEM((BPW, D), jnp.float32),
        pltpu.VMEM((L * PAD,), jnp.float32),
        pltpu.VMEM((BPW,), jnp.float32),
        pltpu.SemaphoreType.DMA,
        pltpu.SemaphoreType.DMA,
    ],
)
def _sc_dual_gather_dot(uidx_hbm, iidx_hbm, user_hbm, item_hbm, out_hbm,
                        uidx_v, iidx_v, urows, irows, tmat, outv,
                        usem, isem):
    wid = lax.axis_index("s") * NC + lax.axis_index("c")
    base = wid * BPW

    pltpu.sync_copy(uidx_hbm.at[wid], uidx_v)
    pltpu.sync_copy(iidx_hbm.at[wid], iidx_v)

    ucopies = [
        pltpu.async_copy(user_hbm.at[uidx_v.at[j]],
                         urows.at[pl.ds(j * CH, CH)], usem)
        for j in range(NCH)
    ]
    icopies = [
        pltpu.async_copy(item_hbm.at[iidx_v.at[j]],
                         irows.at[pl.ds(j * CH, CH)], isem)
        for j in range(NCH)
    ]
    for c in ucopies + icopies:
        c.wait()

    iota = lax.iota(jnp.int32, L)
    gather_idx = [iota * PAD + l for l in range(L)]

    def block_body(blk, _):
        rbase = blk * L
        for j in range(L):
            b = rbase + j
            s = urows[b, pl.ds(0, L)] * irows[b, pl.ds(0, L)]
            for d0 in range(L, D, L):
                s = s + urows[b, pl.ds(d0, L)] * irows[b, pl.ds(d0, L)]
            tmat[pl.ds(j * PAD, L)] = s
        acc = plsc.load_gather(tmat, [gather_idx[0]])
        for l in range(1, L):
            acc = acc + plsc.load_gather(tmat, [gather_idx[l]])
        outv[pl.ds(rbase, L)] = acc
        return 0

    lax.fori_loop(0, BPW // L, block_body, 0)

    pltpu.sync_copy(outv, out_hbm.at[pl.ds(base, BPW)])


def kernel(inputs, user_table, item_table):
    uidx = inputs[:, 0].reshape(NW, NCH, CH)
    iidx = inputs[:, 1].reshape(NW, NCH, CH)
    return _sc_dual_gather_dot(uidx, iidx, user_table, item_table)
